# baseline (speedup 1.0000x reference)
import functools
import math

import jax
import jax.numpy as jnp
from jax import lax
from jax.experimental import pallas as pl
from jax.experimental.pallas import tpu as pltpu
from jax.experimental.pallas import tpu_sc as plsc

D_MODEL = 64
SCALE = math.sqrt(D_MODEL)
NUM_WORKERS = 32
B_TOTAL = 4096 * 200
B_PER_WORKER = B_TOTAL // NUM_WORKERS
CHUNK = 800
N_CHUNKS = B_PER_WORKER // CHUNK
LANES = 16


def _make_kernel():
    mesh = plsc.VectorSubcoreMesh(core_axis_name="c", subcore_axis_name="s")

    @functools.partial(
        pl.kernel,
        out_type=jax.ShapeDtypeStruct((B_TOTAL, D_MODEL), jnp.float32),
        mesh=mesh,
        compiler_params=pltpu.CompilerParams(use_tc_tiling_on_sc=False),
        scratch_types=[
            pltpu.VMEM((CHUNK,), jnp.int32),
            pltpu.VMEM((CHUNK, D_MODEL), jnp.float32),
            pltpu.SemaphoreType.DMA,
        ],
    )
    def gather_scale(idx_hbm, table_hbm, out_hbm, idx_v, rows_v, sem):
        wid = lax.axis_index("s") * 2 + lax.axis_index("c")
        base = wid * B_PER_WORKER

        def chunk_body(c, carry):
            off = base + c * CHUNK
            pltpu.sync_copy(idx_hbm.at[pl.ds(off, CHUNK)], idx_v)
            pltpu.async_copy(table_hbm.at[idx_v], rows_v, sem).wait()

            def scale_body(r, carry2):
                for j in range(D_MODEL // LANES):
                    sl = pl.ds(j * LANES, LANES)
                    rows_v[r, sl] = rows_v[r, sl] * SCALE
                return carry2

            lax.fori_loop(0, CHUNK, scale_body, 0, unroll=4)
            pltpu.sync_copy(rows_v, out_hbm.at[pl.ds(off, CHUNK)])
            return carry

        lax.fori_loop(0, N_CHUNKS, chunk_body, 0)

    return gather_scale


_gather_scale = _make_kernel()


def kernel(x, embed):
    flat_idx = x.reshape(-1)
    out = _gather_scale(flat_idx, embed)
    return out.reshape(x.shape[0], x.shape[1], D_MODEL)


# speedup vs baseline: 1.0699x; 1.0699x over previous
import functools
import math

import jax
import jax.numpy as jnp
from jax import lax
from jax.experimental import pallas as pl
from jax.experimental.pallas import tpu as pltpu
from jax.experimental.pallas import tpu_sc as plsc

D_MODEL = 64
SCALE = math.sqrt(D_MODEL)
NUM_WORKERS = 32
B_TOTAL = 4096 * 200
B_PER_WORKER = B_TOTAL // NUM_WORKERS
CHUNK = 800
N_CHUNKS = B_PER_WORKER // CHUNK
LANES = 16


def _make_kernel():
    mesh = plsc.VectorSubcoreMesh(core_axis_name="c", subcore_axis_name="s")

    @functools.partial(
        pl.kernel,
        out_type=jax.ShapeDtypeStruct((B_TOTAL, D_MODEL), jnp.float32),
        mesh=mesh,
        compiler_params=pltpu.CompilerParams(use_tc_tiling_on_sc=False),
        scratch_types=[
            pltpu.VMEM((B_PER_WORKER,), jnp.int32),
            pltpu.VMEM((CHUNK, D_MODEL), jnp.float32),
            pltpu.VMEM((CHUNK, D_MODEL), jnp.float32),
            pltpu.SemaphoreType.DMA,
            pltpu.SemaphoreType.DMA,
            pltpu.SemaphoreType.DMA,
            pltpu.SemaphoreType.DMA,
        ],
    )
    def gather_scale(idx_hbm, table_hbm, out_hbm,
                     idx_all, rows0, rows1, gs0, gs1, os0, os1):
        wid = lax.axis_index("s") * 2 + lax.axis_index("c")
        base = wid * B_PER_WORKER
        rows = [rows0, rows1]
        gsem = [gs0, gs1]
        osem = [os0, os1]

        pltpu.sync_copy(idx_hbm.at[pl.ds(base, B_PER_WORKER)], idx_all)

        def gather_desc(c, b):
            src = table_hbm.at[idx_all.at[pl.ds(c * CHUNK, CHUNK)]]
            return pltpu.make_async_copy(src, rows[b], gsem[b])

        def out_desc(c, b):
            dst = out_hbm.at[pl.ds(base + c * CHUNK, CHUNK)]
            return pltpu.make_async_copy(rows[b], dst, osem[b])

        gather_desc(0, 0).start()

        def pair_body(p, carry):
            for b in range(2):
                c = p * 2 + b
                gather_desc(c, b).wait()

                @pl.when(c >= 1)
                def _wait_prev_out():
                    out_desc(c - 1, 1 - b).wait()

                @pl.when(c + 1 < N_CHUNKS)
                def _start_next_gather():
                    gather_desc(c + 1, 1 - b).start()

                def scale_body(r, carry2):
                    for j in range(D_MODEL // LANES):
                        sl = pl.ds(j * LANES, LANES)
                        rows[b][r, sl] = rows[b][r, sl] * SCALE
                    return carry2

                lax.fori_loop(0, CHUNK, scale_body, 0, unroll=8)
                out_desc(c, b).start()
            return carry

        lax.fori_loop(0, N_CHUNKS // 2, pair_body, 0)
        out_desc(N_CHUNKS - 1, 1).wait()

    return gather_scale


_gather_scale = _make_kernel()


def kernel(x, embed):
    flat_idx = x.reshape(-1)
    out = _gather_scale(flat_idx, embed)
    return out.reshape(x.shape[0], x.shape[1], D_MODEL)
